# baseline (speedup 1.0000x reference)
import functools

import jax
import jax.numpy as jnp
from jax import lax
from jax.experimental import pallas as pl
from jax.experimental.pallas import tpu as pltpu
from jax.experimental.pallas import tpu_sc as plsc

S, B, E = 8192, 4, 1024
L = 16
NC, NS = 2, 16
NW = NC * NS
RW = S // NW
CH = 4
NCHUNK = RW // CH
NSLOT = 4
D = 2


@functools.partial(
    pl.kernel,
    out_type=jax.ShapeDtypeStruct((S, B, E), jnp.float32),
    mesh=plsc.VectorSubcoreMesh(core_axis_name="c", subcore_axis_name="s"),
    scratch_types=(
        [pltpu.VMEM((CH, B, E), jnp.float32) for _ in range(NSLOT)]
        + [pltpu.VMEM((CH, E), jnp.float32) for _ in range(NSLOT)]
        + [pltpu.SemaphoreType.DMA for _ in range(3 * NSLOT)]
    ),
)
def _sc_add(in_hbm, tab_hbm, out_hbm, *refs):
    in_bufs = refs[0:NSLOT]
    tab_bufs = refs[NSLOT:2 * NSLOT]
    in_sems = refs[2 * NSLOT:3 * NSLOT]
    tab_sems = refs[3 * NSLOT:4 * NSLOT]
    out_sems = refs[4 * NSLOT:5 * NSLOT]

    wid = lax.axis_index("s") * NC + lax.axis_index("c")
    base0 = wid * RW

    def start_in(c, p):
        row = base0 + c * CH
        pltpu.make_async_copy(in_hbm.at[pl.ds(row, CH)], in_bufs[p], in_sems[p]).start()
        pltpu.make_async_copy(tab_hbm.at[pl.ds(row, CH)], tab_bufs[p], tab_sems[p]).start()

    for p in range(D):
        start_in(p, p)

    def outer(c0, carry):
        for k in range(NSLOT):
            c = c0 * NSLOT + k

            if k < D:
                @pl.when(c0 > 0)
                def _wait_out(k=k):
                    pq = (k - D) % NSLOT
                    pltpu.make_async_copy(in_bufs[pq], out_hbm.at[pl.ds(0, CH)], out_sems[pq]).wait()
            else:
                pq = (k - D) % NSLOT
                pltpu.make_async_copy(in_bufs[pq], out_hbm.at[pl.ds(0, CH)], out_sems[pq]).wait()

            @pl.when(c + D < NCHUNK)
            def _start_next_in(c=c, k=k):
                start_in(c + D, (k + D) % NSLOT)

            pltpu.make_async_copy(in_hbm.at[pl.ds(0, CH)], in_bufs[k], in_sems[k]).wait()
            pltpu.make_async_copy(tab_hbm.at[pl.ds(0, CH)], tab_bufs[k], tab_sems[k]).wait()

            def slab(t, cy, k=k):
                r = t // (E // L)
                j = (t % (E // L)) * L
                tab = tab_bufs[k][r, pl.ds(j, L)]
                for b in range(B):
                    plsc.addupdate(in_bufs[k].at[r, b, pl.ds(j, L)], tab)
                return cy

            lax.fori_loop(0, CH * (E // L), slab, 0)

            row = base0 + c * CH
            pltpu.make_async_copy(in_bufs[k], out_hbm.at[pl.ds(row, CH)], out_sems[k]).start()

        return carry

    lax.fori_loop(0, NCHUNK // NSLOT, outer, 0)

    for c in range(NCHUNK - D, NCHUNK):
        pq = c % NSLOT
        pltpu.make_async_copy(in_bufs[pq], out_hbm.at[pl.ds(0, CH)], out_sems[pq]).wait()


def kernel(input, pos_table):
    return _sc_add(input, pos_table)


# speedup vs baseline: 1.0222x; 1.0222x over previous
import functools

import jax
import jax.numpy as jnp
from jax import lax
from jax.experimental import pallas as pl
from jax.experimental.pallas import tpu as pltpu
from jax.experimental.pallas import tpu_sc as plsc

S, B, E = 8192, 4, 1024
L = 16
NC, NS = 2, 16
NW = NC * NS
RW = S // NW
CH = 4
NCHUNK = RW // CH
NSLOT = 4
D = 2


@functools.partial(
    pl.kernel,
    out_type=jax.ShapeDtypeStruct((S, B, E), jnp.float32),
    mesh=plsc.VectorSubcoreMesh(core_axis_name="c", subcore_axis_name="s"),
    scratch_types=(
        [pltpu.VMEM((CH, B, E), jnp.float32) for _ in range(NSLOT)]
        + [pltpu.VMEM((CH, E), jnp.float32) for _ in range(NSLOT)]
        + [pltpu.SemaphoreType.DMA for _ in range(3 * NSLOT)]
    ),
)
def _sc_add(in_hbm, tab_hbm, out_hbm, *refs):
    in_bufs = refs[0:NSLOT]
    tab_bufs = refs[NSLOT:2 * NSLOT]
    in_sems = refs[2 * NSLOT:3 * NSLOT]
    tab_sems = refs[3 * NSLOT:4 * NSLOT]
    out_sems = refs[4 * NSLOT:5 * NSLOT]

    wid = lax.axis_index("s") * NC + lax.axis_index("c")
    base0 = wid * RW

    def start_in(c, p):
        row = base0 + c * CH
        pltpu.make_async_copy(in_hbm.at[pl.ds(row, CH)], in_bufs[p], in_sems[p]).start()
        pltpu.make_async_copy(tab_hbm.at[pl.ds(row, CH)], tab_bufs[p], tab_sems[p]).start()

    for p in range(D):
        start_in(p, p)

    def outer(c0, carry):
        for k in range(NSLOT):
            c = c0 * NSLOT + k

            if k < D:
                @pl.when(c0 > 0)
                def _wait_out(k=k):
                    pq = (k - D) % NSLOT
                    pltpu.make_async_copy(in_bufs[pq], out_hbm.at[pl.ds(0, CH)], out_sems[pq]).wait()
            else:
                pq = (k - D) % NSLOT
                pltpu.make_async_copy(in_bufs[pq], out_hbm.at[pl.ds(0, CH)], out_sems[pq]).wait()

            @pl.when(c + D < NCHUNK)
            def _start_next_in(c=c, k=k):
                start_in(c + D, (k + D) % NSLOT)

            pltpu.make_async_copy(in_hbm.at[pl.ds(0, CH)], in_bufs[k], in_sems[k]).wait()
            pltpu.make_async_copy(tab_hbm.at[pl.ds(0, CH)], tab_bufs[k], tab_sems[k]).wait()

            @plsc.parallel_loop(0, CH * (E // L), unroll=4)
            def _slab(t, k=k):
                r = t // (E // L)
                j = (t % (E // L)) * L
                tab = tab_bufs[k][r, pl.ds(j, L)]
                for b in range(B):
                    plsc.addupdate(in_bufs[k].at[r, b, pl.ds(j, L)], tab)

            row = base0 + c * CH
            pltpu.make_async_copy(in_bufs[k], out_hbm.at[pl.ds(row, CH)], out_sems[k]).start()

        return carry

    lax.fori_loop(0, NCHUNK // NSLOT, outer, 0)

    for c in range(NCHUNK - D, NCHUNK):
        pq = c % NSLOT
        pltpu.make_async_copy(in_bufs[pq], out_hbm.at[pl.ds(0, CH)], out_sems[pq]).wait()


def kernel(input, pos_table):
    return _sc_add(input, pos_table)


# speedup vs baseline: 1.0261x; 1.0038x over previous
import functools

import jax
import jax.numpy as jnp
from jax import lax
from jax.experimental import pallas as pl
from jax.experimental.pallas import tpu as pltpu
from jax.experimental.pallas import tpu_sc as plsc

S, B, E = 8192, 4, 1024
L = 16
NC, NS = 2, 16
NW = NC * NS
RW = S // NW
CH = 2
NCHUNK = RW // CH
NSLOT = 8
D = 4


@functools.partial(
    pl.kernel,
    out_type=jax.ShapeDtypeStruct((S, B, E), jnp.float32),
    mesh=plsc.VectorSubcoreMesh(core_axis_name="c", subcore_axis_name="s"),
    scratch_types=(
        [pltpu.VMEM((CH, B, E), jnp.float32) for _ in range(NSLOT)]
        + [pltpu.VMEM((CH, E), jnp.float32) for _ in range(NSLOT)]
        + [pltpu.SemaphoreType.DMA for _ in range(3 * NSLOT)]
    ),
)
def _sc_add(in_hbm, tab_hbm, out_hbm, *refs):
    in_bufs = refs[0:NSLOT]
    tab_bufs = refs[NSLOT:2 * NSLOT]
    in_sems = refs[2 * NSLOT:3 * NSLOT]
    tab_sems = refs[3 * NSLOT:4 * NSLOT]
    out_sems = refs[4 * NSLOT:5 * NSLOT]

    wid = lax.axis_index("s") * NC + lax.axis_index("c")
    base0 = wid * RW

    def start_in(c, p):
        row = base0 + c * CH
        pltpu.make_async_copy(in_hbm.at[pl.ds(row, CH)], in_bufs[p], in_sems[p]).start()
        pltpu.make_async_copy(tab_hbm.at[pl.ds(row, CH)], tab_bufs[p], tab_sems[p]).start()

    for p in range(D):
        start_in(p, p)

    def outer(c0, carry):
        for k in range(NSLOT):
            c = c0 * NSLOT + k

            if k < D:
                @pl.when(c0 > 0)
                def _wait_out(k=k):
                    pq = (k - D) % NSLOT
                    pltpu.make_async_copy(in_bufs[pq], out_hbm.at[pl.ds(0, CH)], out_sems[pq]).wait()
            else:
                pq = (k - D) % NSLOT
                pltpu.make_async_copy(in_bufs[pq], out_hbm.at[pl.ds(0, CH)], out_sems[pq]).wait()

            @pl.when(c + D < NCHUNK)
            def _start_next_in(c=c, k=k):
                start_in(c + D, (k + D) % NSLOT)

            pltpu.make_async_copy(in_hbm.at[pl.ds(0, CH)], in_bufs[k], in_sems[k]).wait()
            pltpu.make_async_copy(tab_hbm.at[pl.ds(0, CH)], tab_bufs[k], tab_sems[k]).wait()

            @plsc.parallel_loop(0, CH * (E // L), unroll=4)
            def _slab(t, k=k):
                r = t // (E // L)
                j = (t % (E // L)) * L
                tab = tab_bufs[k][r, pl.ds(j, L)]
                for b in range(B):
                    plsc.addupdate(in_bufs[k].at[r, b, pl.ds(j, L)], tab)

            row = base0 + c * CH
            pltpu.make_async_copy(in_bufs[k], out_hbm.at[pl.ds(row, CH)], out_sems[k]).start()

        return carry

    lax.fori_loop(0, NCHUNK // NSLOT, outer, 0)

    for c in range(NCHUNK - D, NCHUNK):
        pq = c % NSLOT
        pltpu.make_async_copy(in_bufs[pq], out_hbm.at[pl.ds(0, CH)], out_sems[pq]).wait()


def kernel(input, pos_table):
    return _sc_add(input, pos_table)
